# Initial kernel scaffold; baseline (speedup 1.0000x reference)
import jax
import jax.numpy as jnp
from jax.experimental import pallas as pl


def kernel(x, edge_index, W1, b1, W2, b2):
    raise NotImplementedError("write your pallas kernel here")


# speedup vs baseline: 6.8311x; 6.8311x over previous
import functools
import jax
import jax.numpy as jnp
from jax import lax
from jax.experimental import pallas as pl
from jax.experimental.pallas import tpu as pltpu
from jax.experimental.pallas import tpu_sc as plsc

N_NODES = 10000
N_EDGES = 160000
R_PAD = 10240
E_PAD = 163840
K = 128
NCH_AGG = 80
NCH_CNT = 40
ROWS_PER_TILE = R_PAD // 16

_mesh = plsc.VectorSubcoreMesh(core_axis_name="c", subcore_axis_name="s")


@functools.partial(
    pl.kernel,
    out_type=jax.ShapeDtypeStruct((2, R_PAD, 128), jnp.float32),
    mesh=_mesh,
    scratch_types=[
        pltpu.VMEM((NCH_CNT, K), jnp.int32),
        pltpu.VMEM((K, 128), jnp.float32),
        pltpu.VMEM_SHARED((R_PAD, 128), jnp.float32),
    ],
)
def _sc_count(dst_hbm, ones_hbm, zeros_hbm, out_hbm, dst_v, ones_v, acc):
    cid = lax.axis_index("c")
    sid = lax.axis_index("s")
    wid = sid * 2 + cid
    rows = pl.ds(sid * ROWS_PER_TILE, ROWS_PER_TILE)

    pltpu.sync_copy(zeros_hbm.at[rows], acc.at[rows])
    pltpu.sync_copy(ones_hbm, ones_v)
    pltpu.sync_copy(dst_hbm.at[wid], dst_v)
    plsc.subcore_barrier()

    def body(c, carry):
        pltpu.sync_copy(ones_v, acc.at[dst_v.at[c]], add=True)
        return carry

    lax.fori_loop(0, NCH_CNT, body, 0)
    plsc.subcore_barrier()
    pltpu.sync_copy(acc.at[rows], out_hbm.at[cid, rows])


def _make_sc_agg(D):
    @functools.partial(
        pl.kernel,
        out_type=jax.ShapeDtypeStruct((2, R_PAD, D), jnp.float32),
        mesh=_mesh,
        scratch_types=[
            pltpu.VMEM((NCH_AGG, K), jnp.int32),
            pltpu.VMEM((NCH_AGG, K), jnp.int32),
            pltpu.VMEM((K, D), jnp.float32),
            pltpu.VMEM_SHARED((R_PAD, D), jnp.float32),
            pltpu.SemaphoreType.DMA,
        ],
    )
    def agg(src2_hbm, dst_hbm, table_hbm, zeros_hbm, out_hbm,
            src_v, dst_v, buf, acc, gsem):
        cid = lax.axis_index("c")
        sid = lax.axis_index("s")
        rows = pl.ds(sid * ROWS_PER_TILE, ROWS_PER_TILE)

        pltpu.sync_copy(zeros_hbm.at[rows], acc.at[rows])
        pltpu.sync_copy(src2_hbm.at[cid, sid], src_v)
        pltpu.sync_copy(dst_hbm.at[sid], dst_v)
        plsc.subcore_barrier()

        def body(c, carry):
            pltpu.async_copy(table_hbm.at[src_v.at[c]], buf, gsem).wait()
            pltpu.sync_copy(buf, acc.at[dst_v.at[c]], add=True)
            return carry

        lax.fori_loop(0, NCH_AGG, body, 0)
        plsc.subcore_barrier()
        pltpu.sync_copy(acc.at[rows], out_hbm.at[cid, rows])

    return agg


_sc_agg128 = _make_sc_agg(128)


@functools.partial(
    pl.kernel,
    out_type=jax.ShapeDtypeStruct((2, R_PAD, 128), jnp.float32),
    mesh=_mesh,
    scratch_types=[
        pltpu.VMEM((NCH_CNT, K), jnp.int32),
        pltpu.VMEM((NCH_CNT, K), jnp.int32),
        pltpu.VMEM((K, 128), jnp.float32),
        pltpu.VMEM_SHARED((R_PAD, 128), jnp.float32),
        pltpu.SemaphoreType.DMA,
    ],
)
def _sc_agg_esplit(src_hbm, dst_hbm, table_hbm, zeros_hbm, out_hbm,
                   src_v, dst_v, buf, acc, gsem):
    cid = lax.axis_index("c")
    sid = lax.axis_index("s")
    wid = sid * 2 + cid
    rows = pl.ds(sid * ROWS_PER_TILE, ROWS_PER_TILE)

    pltpu.sync_copy(zeros_hbm.at[rows], acc.at[rows])
    pltpu.sync_copy(src_hbm.at[wid], src_v)
    pltpu.sync_copy(dst_hbm.at[wid], dst_v)
    plsc.subcore_barrier()

    def body(c, carry):
        pltpu.async_copy(table_hbm.at[src_v.at[c]], buf, gsem).wait()
        pltpu.sync_copy(buf, acc.at[dst_v.at[c]], add=True)
        return carry

    lax.fori_loop(0, NCH_CNT, body, 0)
    plsc.subcore_barrier()
    pltpu.sync_copy(acc.at[rows], out_hbm.at[cid, rows])


_R = 2000
_NR = N_NODES // _R


def _dinv_block(cnt_blk):
    deg = cnt_blk[0] + cnt_blk[1] + 1.0
    return lax.rsqrt(deg)[:, 0:1]


def _tc_pre_body(x_ref, w_ref, cnt_ref, o_ref):
    dinv = _dinv_block(cnt_ref[...])
    h = jnp.dot(x_ref[...], w_ref[...], preferred_element_type=jnp.float32)
    g = h * dinv
    o_ref[0] = g[:, :128]
    o_ref[1] = g[:, 128:]


def _tc_pre(x, W1, cnt):
    return pl.pallas_call(
        _tc_pre_body,
        grid=(_NR,),
        in_specs=[
            pl.BlockSpec((_R, 256), lambda i: (i, 0)),
            pl.BlockSpec((256, 256), lambda i: (0, 0)),
            pl.BlockSpec((2, _R, 128), lambda i: (0, i, 0)),
        ],
        out_specs=pl.BlockSpec((2, _R, 128), lambda i: (0, i, 0)),
        out_shape=jax.ShapeDtypeStruct((2, N_NODES, 128), jnp.float32),
    )(x, W1, cnt)


def _tc_mid_body(agg_ref, g_ref, cnt_ref, b_ref, w_ref, o_ref):
    dinv = _dinv_block(cnt_ref[...])
    w = w_ref[...]
    h0 = jax.nn.relu((agg_ref[0] + g_ref[0]) * dinv + b_ref[0:1, :128])
    h1 = jax.nn.relu((agg_ref[1] + g_ref[1]) * dinv + b_ref[0:1, 128:])
    h2 = (jnp.dot(h0, w[:128, :], preferred_element_type=jnp.float32)
          + jnp.dot(h1, w[128:, :], preferred_element_type=jnp.float32))
    o_ref[...] = h2 * dinv


def _tc_mid(agg1, g1, cnt, b1, W2):
    return pl.pallas_call(
        _tc_mid_body,
        grid=(_NR,),
        in_specs=[
            pl.BlockSpec((2, _R, 128), lambda i: (0, i, 0)),
            pl.BlockSpec((2, _R, 128), lambda i: (0, i, 0)),
            pl.BlockSpec((2, _R, 128), lambda i: (0, i, 0)),
            pl.BlockSpec((1, 256), lambda i: (0, 0)),
            pl.BlockSpec((256, 128), lambda i: (0, 0)),
        ],
        out_specs=pl.BlockSpec((_R, 128), lambda i: (i, 0)),
        out_shape=jax.ShapeDtypeStruct((N_NODES, 128), jnp.float32),
    )(agg1, g1, cnt, b1, W2)


def _tc_post_body(agg_ref, g_ref, cnt_ref, b_ref, o_ref):
    dinv = _dinv_block(cnt_ref[...])
    o_ref[...] = (agg_ref[0] + agg_ref[1] + g_ref[...]) * dinv + b_ref[0:1, :]


def _tc_post(agg2, g2, cnt, b2):
    return pl.pallas_call(
        _tc_post_body,
        grid=(_NR,),
        in_specs=[
            pl.BlockSpec((2, _R, 128), lambda i: (0, i, 0)),
            pl.BlockSpec((_R, 128), lambda i: (i, 0)),
            pl.BlockSpec((2, _R, 128), lambda i: (0, i, 0)),
            pl.BlockSpec((1, 128), lambda i: (0, 0)),
        ],
        out_specs=pl.BlockSpec((_R, 128), lambda i: (i, 0)),
        out_shape=jax.ShapeDtypeStruct((N_NODES, 128), jnp.float32),
    )(agg2, g2, cnt, b2)


def kernel(x, edge_index, W1, b1, W2, b2):
    src = edge_index[0].astype(jnp.int32)
    dst = edge_index[1].astype(jnp.int32)
    pad = E_PAD - N_EDGES
    src_p = jnp.concatenate([src, jnp.zeros((pad,), jnp.int32)])
    dst_p = jnp.concatenate([dst, jnp.full((pad,), N_NODES, jnp.int32)])

    src2 = jnp.stack([src_p, src_p + N_NODES]).reshape(2, 16, NCH_AGG, K)
    dst_agg = dst_p.reshape(16, NCH_AGG, K)
    dst_cnt = dst_p.reshape(32, NCH_CNT, K)

    src_cnt = src_p.reshape(32, NCH_CNT, K)

    ones128 = jnp.ones((K, 128), jnp.float32)
    zeros128 = jnp.zeros((R_PAD, 128), jnp.float32)

    cnt = _sc_count(dst_cnt, ones128, zeros128)[:, :N_NODES, :]

    g1 = _tc_pre(x, W1, cnt)
    agg1 = _sc_agg128(src2, dst_agg, g1.reshape(2 * N_NODES, 128), zeros128)
    g2 = _tc_mid(agg1[:, :N_NODES], g1, cnt, b1.reshape(1, 256), W2)
    agg2 = _sc_agg_esplit(src_cnt, dst_cnt, g2, zeros128)
    z = _tc_post(agg2[:, :N_NODES], g2, cnt, b2.reshape(1, 128))
    return z
